# baseline (speedup 1.0000x reference)
import functools

import jax
import jax.numpy as jnp
from jax import lax
from jax.experimental import pallas as pl
from jax.experimental.pallas import tpu as pltpu
from jax.experimental.pallas import tpu_sc as plsc


def _gather_rows(table, idx):
    _, d = table.shape
    b = idx.shape[0]
    info = plsc.get_sparse_core_info()
    nw = info.num_cores * info.num_subcores
    b_per_w = b // nw
    mesh = plsc.VectorSubcoreMesh(core_axis_name="c", subcore_axis_name="s")

    @functools.partial(
        pl.kernel,
        mesh=mesh,
        out_type=jax.ShapeDtypeStruct((b, d), jnp.float32),
        scratch_types=[
            pltpu.VMEM((b_per_w,), jnp.int32),
            pltpu.VMEM((b_per_w, d), jnp.float32),
            pltpu.SemaphoreType.DMA,
        ],
    )
    def gather_k(table_hbm, idx_hbm, out_hbm, idx_v, rows_v, sem):
        wid = lax.axis_index("s") * info.num_cores + lax.axis_index("c")
        base = wid * b_per_w
        pltpu.sync_copy(idx_hbm.at[pl.ds(base, b_per_w)], idx_v)
        pltpu.async_copy(table_hbm.at[idx_v], rows_v, sem).wait()
        pltpu.sync_copy(rows_v, out_hbm.at[pl.ds(base, b_per_w)])

    return gather_k(table, idx)


def kernel(ob, done, fp, neigh_idx, Wx, bx, Wp, bp, Wm, bm, Wg, a1, a2,
           W_ih, W_hh, b_ih, b_hh, Wa, ba):
    n, do = ob.shape
    na = fp.shape[1]
    deg = neigh_idx.shape[1]
    nfc = Wx.shape[2]
    nh = W_hh.shape[1]
    f = 3 * nfc
    dx = do * (deg + 1)
    dp = na * deg
    blk = 64

    flat_idx = neigh_idx.reshape(-1).astype(jnp.int32)

    pad1 = (-(do + na)) % 128
    table1 = jnp.concatenate(
        [ob, fp, jnp.zeros((n, pad1), jnp.float32)], axis=1)
    g1 = _gather_rows(table1, flat_idx)
    nx = g1[:, :do].reshape(n, deg * do)
    p = g1[:, do:do + na].reshape(n, deg * na)
    xin = jnp.concatenate([ob, nx], axis=1)

    a12 = jnp.concatenate([a1, a2], axis=1)
    fw = f + 64

    def feat_body(xin_ref, p_ref, wx_ref, bx_ref, wp_ref, bp_ref, bm_ref,
                  wg_ref, a12_ref, s_ref, whf_ref):
        hx = jnp.maximum(
            jnp.einsum('ni,nio->no', xin_ref[...], wx_ref[...],
                       preferred_element_type=jnp.float32) + bx_ref[...], 0.0)
        hp = jnp.maximum(
            jnp.einsum('ni,nio->no', p_ref[...], wp_ref[...],
                       preferred_element_type=jnp.float32) + bp_ref[...], 0.0)
        hm = jnp.maximum(bm_ref[...], 0.0)
        s = jnp.concatenate([hx, hp, hm], axis=1)
        wh = jnp.dot(s, wg_ref[...], preferred_element_type=jnp.float32)
        f12 = jnp.dot(wh, a12_ref[...], preferred_element_type=jnp.float32)
        f1 = f12[:, 0:1]
        f2 = f12[:, 1:2]
        s_ref[...] = s
        whf_ref[...] = jnp.concatenate(
            [wh,
             jnp.broadcast_to(f2, (f2.shape[0], 16)),
             jnp.broadcast_to(f1, (f1.shape[0], 16)),
             jnp.zeros((wh.shape[0], fw - f - 32), jnp.float32)], axis=1)

    s_all, whf = pl.pallas_call(
        feat_body,
        grid=(n // blk,),
        in_specs=[
            pl.BlockSpec((blk, dx), lambda i: (i, 0)),
            pl.BlockSpec((blk, dp), lambda i: (i, 0)),
            pl.BlockSpec((blk, dx, nfc), lambda i: (i, 0, 0)),
            pl.BlockSpec((blk, nfc), lambda i: (i, 0)),
            pl.BlockSpec((blk, dp, nfc), lambda i: (i, 0, 0)),
            pl.BlockSpec((blk, nfc), lambda i: (i, 0)),
            pl.BlockSpec((blk, nfc), lambda i: (i, 0)),
            pl.BlockSpec((f, f), lambda i: (0, 0)),
            pl.BlockSpec((f, 2), lambda i: (0, 0)),
        ],
        out_specs=[
            pl.BlockSpec((blk, f), lambda i: (i, 0)),
            pl.BlockSpec((blk, fw), lambda i: (i, 0)),
        ],
        out_shape=[
            jax.ShapeDtypeStruct((n, f), jnp.float32),
            jax.ShapeDtypeStruct((n, fw), jnp.float32),
        ],
    )(xin, p, Wx, bx, Wp, bp, bm, Wg, a12)

    g2 = _gather_rows(whf, flat_idx).reshape(n, deg, fw)

    w_iht = W_ih.T
    bih2 = b_ih.reshape(1, -1)
    bhh2 = b_hh.reshape(1, -1)

    def head_body(s_ref, whf_ref, g2_ref, wiht_ref, bih_ref, bhh_ref,
                  wa_ref, ba_ref, out_ref):
        whf_b = whf_ref[...]
        wh_s = whf_b[:, :f]
        f2s = jnp.max(whf_b[:, f:f + 16], axis=1, keepdims=True)
        f1 = jnp.max(whf_b[:, f + 16:f + 32], axis=1, keepdims=True)
        g2_b = g2_ref[...]
        f2n = jnp.max(g2_b[:, :, f:f + 16], axis=2)
        e = f1 + jnp.concatenate([f2s, f2n], axis=1)
        e = jnp.where(e > 0, e, 0.2 * e)
        m = jnp.max(e, axis=1, keepdims=True)
        ex = jnp.exp(e - m)
        att = ex / jnp.sum(ex, axis=1, keepdims=True)
        acc = att[:, 0:1] * wh_s
        for k in range(deg):
            acc = acc + att[:, k + 1:k + 2] * g2_b[:, k, :f]
        gat = jnp.where(acc > 0, acc, jnp.exp(acc) - 1.0)
        s2 = s_ref[...] + gat
        gates = (jnp.dot(s2, wiht_ref[...], preferred_element_type=jnp.float32)
                 + bih_ref[...] + bhh_ref[...])
        i_g = jax.nn.sigmoid(gates[:, :nh])
        g_g = jnp.tanh(gates[:, 2 * nh:3 * nh])
        o_g = jax.nn.sigmoid(gates[:, 3 * nh:4 * nh])
        h = o_g * jnp.tanh(i_g * g_g)
        logits = jnp.einsum('nh,nha->na', h, wa_ref[...],
                            preferred_element_type=jnp.float32) + ba_ref[...]
        mx = jnp.max(logits, axis=1, keepdims=True)
        exl = jnp.exp(logits - mx)
        out_ref[...] = exl / jnp.sum(exl, axis=1, keepdims=True)

    probs = pl.pallas_call(
        head_body,
        grid=(n // blk,),
        in_specs=[
            pl.BlockSpec((blk, f), lambda i: (i, 0)),
            pl.BlockSpec((blk, fw), lambda i: (i, 0)),
            pl.BlockSpec((blk, deg, fw), lambda i: (i, 0, 0)),
            pl.BlockSpec((f, 4 * nh), lambda i: (0, 0)),
            pl.BlockSpec((1, 4 * nh), lambda i: (0, 0)),
            pl.BlockSpec((1, 4 * nh), lambda i: (0, 0)),
            pl.BlockSpec((blk, nh, na), lambda i: (i, 0, 0)),
            pl.BlockSpec((blk, na), lambda i: (i, 0)),
        ],
        out_specs=pl.BlockSpec((blk, na), lambda i: (i, 0)),
        out_shape=jax.ShapeDtypeStruct((n, na), jnp.float32),
    )(s_all, whf, g2, w_iht, bih2, bhh2, Wa, ba)

    return probs


# speedup vs baseline: 3.4704x; 3.4704x over previous
import functools

import jax
import jax.numpy as jnp
from jax import lax
from jax.experimental import pallas as pl
from jax.experimental.pallas import tpu as pltpu
from jax.experimental.pallas import tpu_sc as plsc


def _gather_rows(table, idx):
    _, d = table.shape
    b = idx.shape[0]
    info = plsc.get_sparse_core_info()
    nw = info.num_cores * info.num_subcores
    b_per_w = b // nw
    mesh = plsc.VectorSubcoreMesh(core_axis_name="c", subcore_axis_name="s")

    @functools.partial(
        pl.kernel,
        mesh=mesh,
        out_type=jax.ShapeDtypeStruct((b, d), jnp.float32),
        scratch_types=[
            pltpu.VMEM((b_per_w,), jnp.int32),
            pltpu.VMEM((b_per_w, d), jnp.float32),
            pltpu.SemaphoreType.DMA,
        ],
    )
    def gather_k(table_hbm, idx_hbm, out_hbm, idx_v, rows_v, sem):
        wid = lax.axis_index("s") * info.num_cores + lax.axis_index("c")
        base = wid * b_per_w
        pltpu.sync_copy(idx_hbm.at[pl.ds(base, b_per_w)], idx_v)
        pltpu.async_copy(table_hbm.at[idx_v], rows_v, sem).wait()
        pltpu.sync_copy(rows_v, out_hbm.at[pl.ds(base, b_per_w)])

    return gather_k(table, idx)


def kernel(ob, done, fp, neigh_idx, Wx, bx, Wp, bp, Wm, bm, Wg, a1, a2,
           W_ih, W_hh, b_ih, b_hh, Wa, ba):
    n, do = ob.shape
    na = fp.shape[1]
    deg = neigh_idx.shape[1]
    nfc = Wx.shape[2]
    nh = W_hh.shape[1]
    f = 3 * nfc
    dx = do * (deg + 1)
    dp = na * deg
    bn = 128
    ic = 64
    nck = dx // ic
    gn = n // bn

    flat_idx = neigh_idx.reshape(-1).astype(jnp.int32)

    pad1 = (-(do + na)) % 128
    table1 = jnp.concatenate(
        [ob, fp, jnp.zeros((n, pad1), jnp.float32)], axis=1)
    g1 = _gather_rows(table1, flat_idx)

    nxT = g1[:, :do].reshape(n, deg * do).T
    pT = g1[:, do:do + na].reshape(n, deg * na).T
    xinT = jnp.concatenate([ob.T, nxT], axis=0)
    WxT = Wx.transpose(1, 2, 0)
    WpT = Wp.transpose(1, 2, 0)
    WaT = Wa.transpose(1, 2, 0)
    bxT, bpT, bmT, baT = bx.T, bp.T, bm.T, ba.T
    idxT = neigh_idx.T.astype(jnp.int32)
    w_iht = W_ih.T
    bihC = (b_ih + b_hh).reshape(-1, 1)

    def feat_body(xinT_ref, pT_ref, wxT_ref, bxT_ref, wpT_ref, bpT_ref,
                  bmT_ref, wg_ref, sT_ref, whT_ref, accx, accp):
        j = pl.program_id(1)

        @pl.when(j == 0)
        def _init():
            acc = jnp.zeros((nfc, bn), jnp.float32)
            for r in range(dp):
                acc = acc + wpT_ref[r] * pT_ref[r:r + 1, :]
            accp[...] = acc
            accx[...] = jnp.zeros((nfc, bn), jnp.float32)

        acc = accx[...]
        for r in range(ic):
            acc = acc + wxT_ref[r] * xinT_ref[r:r + 1, :]
        accx[...] = acc

        @pl.when(j == nck - 1)
        def _finalize():
            hx = jnp.maximum(accx[...] + bxT_ref[...], 0.0)
            hp = jnp.maximum(accp[...] + bpT_ref[...], 0.0)
            hm = jnp.maximum(bmT_ref[...], 0.0)
            sT = jnp.concatenate([hx, hp, hm], axis=0)
            sT_ref[...] = sT
            whT_ref[...] = lax.dot_general(
                wg_ref[...], sT, (((0,), (0,)), ((), ())),
                preferred_element_type=jnp.float32)

    sT, whT = pl.pallas_call(
        feat_body,
        grid=(gn, nck),
        in_specs=[
            pl.BlockSpec((ic, bn), lambda i, j: (j, i)),
            pl.BlockSpec((dp, bn), lambda i, j: (0, i)),
            pl.BlockSpec((ic, nfc, bn), lambda i, j: (j, 0, i)),
            pl.BlockSpec((nfc, bn), lambda i, j: (0, i)),
            pl.BlockSpec((dp, nfc, bn), lambda i, j: (0, 0, i)),
            pl.BlockSpec((nfc, bn), lambda i, j: (0, i)),
            pl.BlockSpec((nfc, bn), lambda i, j: (0, i)),
            pl.BlockSpec((f, f), lambda i, j: (0, 0)),
        ],
        out_specs=[
            pl.BlockSpec((f, bn), lambda i, j: (0, i)),
            pl.BlockSpec((f, bn), lambda i, j: (0, i)),
        ],
        out_shape=[
            jax.ShapeDtypeStruct((f, n), jnp.float32),
            jax.ShapeDtypeStruct((f, n), jnp.float32),
        ],
        scratch_shapes=[
            pltpu.VMEM((nfc, bn), jnp.float32),
            pltpu.VMEM((nfc, bn), jnp.float32),
        ],
    )(xinT, pT, WxT, bxT, WpT, bpT, bmT, Wg)

    def head_body(sT_ref, whT_full_ref, whT_blk_ref, idxT_ref, a1_ref,
                  a2_ref, wiht_ref, bih_ref, waT_ref, baT_ref, out_ref):
        i = pl.program_id(0)
        whT_full = whT_full_ref[...]
        f2c = lax.dot_general(whT_full, a2_ref[...],
                              (((0,), (0,)), ((), ())),
                              preferred_element_type=jnp.float32)
        f1r = lax.dot_general(a1_ref[...], whT_blk_ref[...],
                              (((0,), (0,)), ((), ())),
                              preferred_element_type=jnp.float32)
        e = f2c + f1r
        e = jnp.where(e > 0, e, 0.2 * e)
        jsub = lax.broadcasted_iota(jnp.int32, (n, bn), 0)
        adj = jsub == (i * bn + lax.broadcasted_iota(jnp.int32, (n, bn), 1))
        for k in range(deg):
            adj = adj | (jsub == idxT_ref[k:k + 1, :])
        e = jnp.where(adj, e, jnp.float32(-9e15))
        m = jnp.max(e, axis=0, keepdims=True)
        ex = jnp.exp(e - m)
        att = ex / jnp.sum(ex, axis=0, keepdims=True)
        gat = lax.dot_general(whT_full, att, (((1,), (0,)), ((), ())),
                              preferred_element_type=jnp.float32)
        gat = jnp.where(gat > 0, gat, jnp.exp(gat) - 1.0)
        s2 = sT_ref[...] + gat
        gates = lax.dot_general(wiht_ref[...], s2, (((0,), (0,)), ((), ())),
                                preferred_element_type=jnp.float32)
        gates = gates + bih_ref[...]
        i_g = jax.nn.sigmoid(gates[:nh])
        g_g = jnp.tanh(gates[2 * nh:3 * nh])
        o_g = jax.nn.sigmoid(gates[3 * nh:4 * nh])
        h = o_g * jnp.tanh(i_g * g_g)
        acc = baT_ref[...].astype(jnp.float32)
        for r in range(nh):
            acc = acc + waT_ref[r] * h[r:r + 1, :]
        mx = jnp.max(acc, axis=0, keepdims=True)
        exl = jnp.exp(acc - mx)
        out_ref[...] = exl / jnp.sum(exl, axis=0, keepdims=True)

    probsT = pl.pallas_call(
        head_body,
        grid=(gn,),
        in_specs=[
            pl.BlockSpec((f, bn), lambda i: (0, i)),
            pl.BlockSpec((f, n), lambda i: (0, 0)),
            pl.BlockSpec((f, bn), lambda i: (0, i)),
            pl.BlockSpec((deg, bn), lambda i: (0, i)),
            pl.BlockSpec((f, 1), lambda i: (0, 0)),
            pl.BlockSpec((f, 1), lambda i: (0, 0)),
            pl.BlockSpec((f, 4 * nh), lambda i: (0, 0)),
            pl.BlockSpec((4 * nh, 1), lambda i: (0, 0)),
            pl.BlockSpec((nh, na, bn), lambda i: (0, 0, i)),
            pl.BlockSpec((na, bn), lambda i: (0, i)),
        ],
        out_specs=pl.BlockSpec((na, bn), lambda i: (0, i)),
        out_shape=jax.ShapeDtypeStruct((na, n), jnp.float32),
    )(sT, whT, whT, idxT, a1, a2, w_iht, bihC, WaT, baT)

    return probsT.T


# speedup vs baseline: 4.1336x; 1.1911x over previous
import functools

import jax
import jax.numpy as jnp
from jax import lax
from jax.experimental import pallas as pl
from jax.experimental.pallas import tpu as pltpu
from jax.experimental.pallas import tpu_sc as plsc


def _gather_rows(table, idx):
    _, d = table.shape
    b = idx.shape[0]
    info = plsc.get_sparse_core_info()
    nw = info.num_cores * info.num_subcores
    b_per_w = b // nw
    mesh = plsc.VectorSubcoreMesh(core_axis_name="c", subcore_axis_name="s")

    @functools.partial(
        pl.kernel,
        mesh=mesh,
        out_type=jax.ShapeDtypeStruct((b, d), jnp.float32),
        scratch_types=[
            pltpu.VMEM((b_per_w,), jnp.int32),
            pltpu.VMEM((b_per_w, d), jnp.float32),
            pltpu.SemaphoreType.DMA,
        ],
    )
    def gather_k(table_hbm, idx_hbm, out_hbm, idx_v, rows_v, sem):
        wid = lax.axis_index("s") * info.num_cores + lax.axis_index("c")
        base = wid * b_per_w
        pltpu.sync_copy(idx_hbm.at[pl.ds(base, b_per_w)], idx_v)
        pltpu.async_copy(table_hbm.at[idx_v], rows_v, sem).wait()
        pltpu.sync_copy(rows_v, out_hbm.at[pl.ds(base, b_per_w)])

    return gather_k(table, idx)


def kernel(ob, done, fp, neigh_idx, Wx, bx, Wp, bp, Wm, bm, Wg, a1, a2,
           W_ih, W_hh, b_ih, b_hh, Wa, ba):
    n, do = ob.shape
    na = fp.shape[1]
    deg = neigh_idx.shape[1]
    nfc = Wx.shape[2]
    nh = W_hh.shape[1]
    f = 3 * nfc
    dx = do * (deg + 1)
    dp = na * deg
    bn = 256
    ic = 64
    nck = dx // ic
    gn = n // bn

    flat_idx = neigh_idx.reshape(-1).astype(jnp.int32)

    pad1 = (-(do + na)) % 128
    table1 = jnp.concatenate(
        [ob, fp, jnp.zeros((n, pad1), jnp.float32)], axis=1)
    g1 = _gather_rows(table1, flat_idx)

    nxT = g1[:, :do].reshape(n, deg * do).T
    pT = g1[:, do:do + na].reshape(n, deg * na).T
    xinT = jnp.concatenate([ob.T, nxT], axis=0)
    WxT = Wx.transpose(1, 2, 0)
    WpT = Wp.transpose(1, 2, 0)
    WaT = Wa.transpose(1, 2, 0)
    bxT, bpT, bmT, baT = bx.T, bp.T, bm.T, ba.T
    idxT = neigh_idx.T.astype(jnp.int32)
    w_iht = W_ih.T
    bihC = (b_ih + b_hh).reshape(-1, 1)

    def feat_body(xinT_ref, pT_ref, wxT_ref, bxT_ref, wpT_ref, bpT_ref,
                  bmT_ref, wg_ref, sT_ref, whT_ref, accx, accp):
        j = pl.program_id(1)

        @pl.when(j == 0)
        def _init():
            acc = jnp.zeros((nfc, bn), jnp.float32)
            for r in range(dp):
                acc = acc + wpT_ref[r] * pT_ref[r:r + 1, :]
            accp[...] = acc
            accx[...] = jnp.zeros((nfc, bn), jnp.float32)

        acc = accx[...]
        for r in range(ic):
            acc = acc + wxT_ref[r] * xinT_ref[r:r + 1, :]
        accx[...] = acc

        @pl.when(j == nck - 1)
        def _finalize():
            hx = jnp.maximum(accx[...] + bxT_ref[...], 0.0)
            hp = jnp.maximum(accp[...] + bpT_ref[...], 0.0)
            hm = jnp.maximum(bmT_ref[...], 0.0)
            sT = jnp.concatenate([hx, hp, hm], axis=0)
            sT_ref[...] = sT
            whT_ref[...] = lax.dot_general(
                wg_ref[...], sT, (((0,), (0,)), ((), ())),
                preferred_element_type=jnp.float32)

    sT, whT = pl.pallas_call(
        feat_body,
        grid=(gn, nck),
        in_specs=[
            pl.BlockSpec((ic, bn), lambda i, j: (j, i)),
            pl.BlockSpec((dp, bn), lambda i, j: (0, i)),
            pl.BlockSpec((ic, nfc, bn), lambda i, j: (j, 0, i)),
            pl.BlockSpec((nfc, bn), lambda i, j: (0, i)),
            pl.BlockSpec((dp, nfc, bn), lambda i, j: (0, 0, i)),
            pl.BlockSpec((nfc, bn), lambda i, j: (0, i)),
            pl.BlockSpec((nfc, bn), lambda i, j: (0, i)),
            pl.BlockSpec((f, f), lambda i, j: (0, 0)),
        ],
        out_specs=[
            pl.BlockSpec((f, bn), lambda i, j: (0, i)),
            pl.BlockSpec((f, bn), lambda i, j: (0, i)),
        ],
        out_shape=[
            jax.ShapeDtypeStruct((f, n), jnp.float32),
            jax.ShapeDtypeStruct((f, n), jnp.float32),
        ],
        scratch_shapes=[
            pltpu.VMEM((nfc, bn), jnp.float32),
            pltpu.VMEM((nfc, bn), jnp.float32),
        ],
    )(xinT, pT, WxT, bxT, WpT, bpT, bmT, Wg)

    def head_body(sT_ref, whT_full_ref, whT_blk_ref, idxT_ref, a1_ref,
                  a2_ref, wiht_ref, bih_ref, waT_ref, baT_ref, out_ref):
        i = pl.program_id(0)
        whT_full = whT_full_ref[...]
        f2c = lax.dot_general(whT_full, a2_ref[...],
                              (((0,), (0,)), ((), ())),
                              preferred_element_type=jnp.float32)
        f1r = lax.dot_general(a1_ref[...], whT_blk_ref[...],
                              (((0,), (0,)), ((), ())),
                              preferred_element_type=jnp.float32)
        e = f2c + f1r
        e = jnp.where(e > 0, e, 0.2 * e)
        jsub = lax.broadcasted_iota(jnp.int32, (n, bn), 0)
        adj = jsub == (i * bn + lax.broadcasted_iota(jnp.int32, (n, bn), 1))
        for k in range(deg):
            adj = adj | (jsub == idxT_ref[k:k + 1, :])
        e = jnp.where(adj, e, jnp.float32(-9e15))
        m = jnp.max(e, axis=0, keepdims=True)
        ex = jnp.exp(e - m)
        att = ex / jnp.sum(ex, axis=0, keepdims=True)
        gat = lax.dot_general(whT_full, att, (((1,), (0,)), ((), ())),
                              preferred_element_type=jnp.float32)
        gat = jnp.where(gat > 0, gat, jnp.exp(gat) - 1.0)
        s2 = sT_ref[...] + gat
        gates = lax.dot_general(wiht_ref[...], s2, (((0,), (0,)), ((), ())),
                                preferred_element_type=jnp.float32)
        gates = gates + bih_ref[...]
        i_g = jax.nn.sigmoid(gates[:nh])
        g_g = jnp.tanh(gates[2 * nh:3 * nh])
        o_g = jax.nn.sigmoid(gates[3 * nh:4 * nh])
        h = o_g * jnp.tanh(i_g * g_g)
        acc = baT_ref[...].astype(jnp.float32)
        for r in range(nh):
            acc = acc + waT_ref[r] * h[r:r + 1, :]
        mx = jnp.max(acc, axis=0, keepdims=True)
        exl = jnp.exp(acc - mx)
        out_ref[...] = exl / jnp.sum(exl, axis=0, keepdims=True)

    probsT = pl.pallas_call(
        head_body,
        grid=(gn,),
        in_specs=[
            pl.BlockSpec((f, bn), lambda i: (0, i)),
            pl.BlockSpec((f, n), lambda i: (0, 0)),
            pl.BlockSpec((f, bn), lambda i: (0, i)),
            pl.BlockSpec((deg, bn), lambda i: (0, i)),
            pl.BlockSpec((f, 1), lambda i: (0, 0)),
            pl.BlockSpec((f, 1), lambda i: (0, 0)),
            pl.BlockSpec((f, 4 * nh), lambda i: (0, 0)),
            pl.BlockSpec((4 * nh, 1), lambda i: (0, 0)),
            pl.BlockSpec((nh, na, bn), lambda i: (0, 0, i)),
            pl.BlockSpec((na, bn), lambda i: (0, i)),
        ],
        out_specs=pl.BlockSpec((na, bn), lambda i: (0, i)),
        out_shape=jax.ShapeDtypeStruct((na, n), jnp.float32),
    )(sT, whT, whT, idxT, a1, a2, w_iht, bihC, WaT, baT)

    return probsT.T


# speedup vs baseline: 4.4605x; 1.0791x over previous
import functools

import jax
import jax.numpy as jnp
from jax import lax
from jax.experimental import pallas as pl
from jax.experimental.pallas import tpu as pltpu
from jax.experimental.pallas import tpu_sc as plsc


def _gather_rows(table, idx):
    _, d = table.shape
    b = idx.shape[0]
    info = plsc.get_sparse_core_info()
    nw = info.num_cores * info.num_subcores
    b_per_w = b // nw
    mesh = plsc.VectorSubcoreMesh(core_axis_name="c", subcore_axis_name="s")

    @functools.partial(
        pl.kernel,
        mesh=mesh,
        out_type=jax.ShapeDtypeStruct((b, d), jnp.float32),
        scratch_types=[
            pltpu.VMEM((b_per_w,), jnp.int32),
            pltpu.VMEM((b_per_w, d), jnp.float32),
            pltpu.SemaphoreType.DMA,
        ],
    )
    def gather_k(table_hbm, idx_hbm, out_hbm, idx_v, rows_v, sem):
        wid = lax.axis_index("s") * info.num_cores + lax.axis_index("c")
        base = wid * b_per_w
        pltpu.sync_copy(idx_hbm.at[pl.ds(base, b_per_w)], idx_v)
        pltpu.async_copy(table_hbm.at[idx_v], rows_v, sem).wait()
        pltpu.sync_copy(rows_v, out_hbm.at[pl.ds(base, b_per_w)])

    return gather_k(table, idx)


def kernel(ob, done, fp, neigh_idx, Wx, bx, Wp, bp, Wm, bm, Wg, a1, a2,
           W_ih, W_hh, b_ih, b_hh, Wa, ba):
    n, do = ob.shape
    na = fp.shape[1]
    deg = neigh_idx.shape[1]
    nfc = Wx.shape[2]
    nh = W_hh.shape[1]
    f = 3 * nfc
    dx = do * (deg + 1)
    dp = na * deg
    bn = 512
    ic = 64
    nck = dx // ic
    gn = n // bn

    flat_idx = neigh_idx.reshape(-1).astype(jnp.int32)

    pad1 = (-(do + na)) % 128
    table1 = jnp.concatenate(
        [ob, fp, jnp.zeros((n, pad1), jnp.float32)], axis=1)
    g1 = _gather_rows(table1, flat_idx)

    nxT = g1[:, :do].reshape(n, deg * do).T
    pT = g1[:, do:do + na].reshape(n, deg * na).T
    xinT = jnp.concatenate([ob.T, nxT], axis=0)
    WxT = Wx.transpose(1, 2, 0)
    WpT = Wp.transpose(1, 2, 0)
    WaT = Wa.transpose(1, 2, 0)
    bxT, bpT, bmT, baT = bx.T, bp.T, bm.T, ba.T
    idxT = neigh_idx.T.astype(jnp.int32)
    w_iht = W_ih.T
    bihC = (b_ih + b_hh).reshape(-1, 1)

    def feat_body(xinT_ref, pT_ref, wxT_ref, bxT_ref, wpT_ref, bpT_ref,
                  bmT_ref, wg_ref, sT_ref, whT_ref, accx, accp):
        j = pl.program_id(1)

        @pl.when(j == 0)
        def _init():
            acc = jnp.zeros((nfc, bn), jnp.float32)
            for r in range(dp):
                acc = acc + wpT_ref[r] * pT_ref[r:r + 1, :]
            accp[...] = acc
            accx[...] = jnp.zeros((nfc, bn), jnp.float32)

        acc = accx[...]
        for r in range(ic):
            acc = acc + wxT_ref[r] * xinT_ref[r:r + 1, :]
        accx[...] = acc

        @pl.when(j == nck - 1)
        def _finalize():
            hx = jnp.maximum(accx[...] + bxT_ref[...], 0.0)
            hp = jnp.maximum(accp[...] + bpT_ref[...], 0.0)
            hm = jnp.maximum(bmT_ref[...], 0.0)
            sT = jnp.concatenate([hx, hp, hm], axis=0)
            sT_ref[...] = sT
            whT_ref[...] = lax.dot_general(
                wg_ref[...], sT, (((0,), (0,)), ((), ())),
                preferred_element_type=jnp.float32)

    sT, whT = pl.pallas_call(
        feat_body,
        grid=(gn, nck),
        in_specs=[
            pl.BlockSpec((ic, bn), lambda i, j: (j, i)),
            pl.BlockSpec((dp, bn), lambda i, j: (0, i)),
            pl.BlockSpec((ic, nfc, bn), lambda i, j: (j, 0, i)),
            pl.BlockSpec((nfc, bn), lambda i, j: (0, i)),
            pl.BlockSpec((dp, nfc, bn), lambda i, j: (0, 0, i)),
            pl.BlockSpec((nfc, bn), lambda i, j: (0, i)),
            pl.BlockSpec((nfc, bn), lambda i, j: (0, i)),
            pl.BlockSpec((f, f), lambda i, j: (0, 0)),
        ],
        out_specs=[
            pl.BlockSpec((f, bn), lambda i, j: (0, i)),
            pl.BlockSpec((f, bn), lambda i, j: (0, i)),
        ],
        out_shape=[
            jax.ShapeDtypeStruct((f, n), jnp.float32),
            jax.ShapeDtypeStruct((f, n), jnp.float32),
        ],
        scratch_shapes=[
            pltpu.VMEM((nfc, bn), jnp.float32),
            pltpu.VMEM((nfc, bn), jnp.float32),
        ],
    )(xinT, pT, WxT, bxT, WpT, bpT, bmT, Wg)

    def head_body(sT_ref, whT_full_ref, whT_blk_ref, idxT_ref, a1_ref,
                  a2_ref, wiht_ref, bih_ref, waT_ref, baT_ref, out_ref):
        i = pl.program_id(0)
        whT_full = whT_full_ref[...]
        f2c = lax.dot_general(whT_full, a2_ref[...],
                              (((0,), (0,)), ((), ())),
                              preferred_element_type=jnp.float32)
        f1r = lax.dot_general(a1_ref[...], whT_blk_ref[...],
                              (((0,), (0,)), ((), ())),
                              preferred_element_type=jnp.float32)
        e = f2c + f1r
        e = jnp.where(e > 0, e, 0.2 * e)
        jsub = lax.broadcasted_iota(jnp.int32, (n, bn), 0)
        adj = jsub == (i * bn + lax.broadcasted_iota(jnp.int32, (n, bn), 1))
        for k in range(deg):
            adj = adj | (jsub == idxT_ref[k:k + 1, :])
        e = jnp.where(adj, e, jnp.float32(-9e15))
        m = jnp.max(e, axis=0, keepdims=True)
        ex = jnp.exp(e - m)
        att = ex / jnp.sum(ex, axis=0, keepdims=True)
        gat = lax.dot_general(whT_full, att, (((1,), (0,)), ((), ())),
                              preferred_element_type=jnp.float32)
        gat = jnp.where(gat > 0, gat, jnp.exp(gat) - 1.0)
        s2 = sT_ref[...] + gat
        gates = lax.dot_general(wiht_ref[...], s2, (((0,), (0,)), ((), ())),
                                preferred_element_type=jnp.float32)
        gates = gates + bih_ref[...]
        i_g = jax.nn.sigmoid(gates[:nh])
        g_g = jnp.tanh(gates[2 * nh:3 * nh])
        o_g = jax.nn.sigmoid(gates[3 * nh:4 * nh])
        h = o_g * jnp.tanh(i_g * g_g)
        acc = baT_ref[...].astype(jnp.float32)
        for r in range(nh):
            acc = acc + waT_ref[r] * h[r:r + 1, :]
        mx = jnp.max(acc, axis=0, keepdims=True)
        exl = jnp.exp(acc - mx)
        out_ref[...] = exl / jnp.sum(exl, axis=0, keepdims=True)

    probsT = pl.pallas_call(
        head_body,
        grid=(gn,),
        in_specs=[
            pl.BlockSpec((f, bn), lambda i: (0, i)),
            pl.BlockSpec((f, n), lambda i: (0, 0)),
            pl.BlockSpec((f, bn), lambda i: (0, i)),
            pl.BlockSpec((deg, bn), lambda i: (0, i)),
            pl.BlockSpec((f, 1), lambda i: (0, 0)),
            pl.BlockSpec((f, 1), lambda i: (0, 0)),
            pl.BlockSpec((f, 4 * nh), lambda i: (0, 0)),
            pl.BlockSpec((4 * nh, 1), lambda i: (0, 0)),
            pl.BlockSpec((nh, na, bn), lambda i: (0, 0, i)),
            pl.BlockSpec((na, bn), lambda i: (0, i)),
        ],
        out_specs=pl.BlockSpec((na, bn), lambda i: (0, i)),
        out_shape=jax.ShapeDtypeStruct((na, n), jnp.float32),
    )(sT, whT, whT, idxT, a1, a2, w_iht, bihC, WaT, baT)

    return probsT.T


# speedup vs baseline: 4.5044x; 1.0098x over previous
import functools

import jax
import jax.numpy as jnp
from jax import lax
from jax.experimental import pallas as pl
from jax.experimental.pallas import tpu as pltpu
from jax.experimental.pallas import tpu_sc as plsc


def _gather_rows(table, idx):
    _, d = table.shape
    b = idx.shape[0]
    info = plsc.get_sparse_core_info()
    nw = info.num_cores * info.num_subcores
    b_per_w = b // nw
    mesh = plsc.VectorSubcoreMesh(core_axis_name="c", subcore_axis_name="s")

    @functools.partial(
        pl.kernel,
        mesh=mesh,
        out_type=jax.ShapeDtypeStruct((b, d), jnp.float32),
        scratch_types=[
            pltpu.VMEM((b_per_w,), jnp.int32),
            pltpu.VMEM((b_per_w, d), jnp.float32),
            pltpu.SemaphoreType.DMA,
        ],
    )
    def gather_k(table_hbm, idx_hbm, out_hbm, idx_v, rows_v, sem):
        wid = lax.axis_index("s") * info.num_cores + lax.axis_index("c")
        base = wid * b_per_w
        pltpu.sync_copy(idx_hbm.at[pl.ds(base, b_per_w)], idx_v)
        pltpu.async_copy(table_hbm.at[idx_v], rows_v, sem).wait()
        pltpu.sync_copy(rows_v, out_hbm.at[pl.ds(base, b_per_w)])

    return gather_k(table, idx)


def kernel(ob, done, fp, neigh_idx, Wx, bx, Wp, bp, Wm, bm, Wg, a1, a2,
           W_ih, W_hh, b_ih, b_hh, Wa, ba):
    n, do = ob.shape
    na = fp.shape[1]
    deg = neigh_idx.shape[1]
    nfc = Wx.shape[2]
    nh = W_hh.shape[1]
    f = 3 * nfc
    dx = do * (deg + 1)
    dp = na * deg
    bn = 1024
    ic = 64
    nck = dx // ic
    gn = n // bn

    flat_idx = neigh_idx.reshape(-1).astype(jnp.int32)

    pad1 = (-(do + na)) % 128
    table1 = jnp.concatenate(
        [ob, fp, jnp.zeros((n, pad1), jnp.float32)], axis=1)
    g1 = _gather_rows(table1, flat_idx)

    nxT = g1[:, :do].reshape(n, deg * do).T
    pT = g1[:, do:do + na].reshape(n, deg * na).T
    xinT = jnp.concatenate([ob.T, nxT], axis=0)
    WxT = Wx.transpose(1, 2, 0)
    WpT = Wp.transpose(1, 2, 0)
    WaT = Wa.transpose(1, 2, 0)
    bxT, bpT, bmT, baT = bx.T, bp.T, bm.T, ba.T
    idxT = neigh_idx.T.astype(jnp.int32)
    w_iht = W_ih.T
    bihC = (b_ih + b_hh).reshape(-1, 1)

    def feat_body(xinT_ref, pT_ref, wxT_ref, bxT_ref, wpT_ref, bpT_ref,
                  bmT_ref, wg_ref, sT_ref, whT_ref, accx, accp):
        j = pl.program_id(1)

        @pl.when(j == 0)
        def _init():
            acc = jnp.zeros((nfc, bn), jnp.float32)
            for r in range(dp):
                acc = acc + wpT_ref[r] * pT_ref[r:r + 1, :]
            accp[...] = acc
            accx[...] = jnp.zeros((nfc, bn), jnp.float32)

        acc = accx[...]
        for r in range(ic):
            acc = acc + wxT_ref[r] * xinT_ref[r:r + 1, :]
        accx[...] = acc

        @pl.when(j == nck - 1)
        def _finalize():
            hx = jnp.maximum(accx[...] + bxT_ref[...], 0.0)
            hp = jnp.maximum(accp[...] + bpT_ref[...], 0.0)
            hm = jnp.maximum(bmT_ref[...], 0.0)
            sT = jnp.concatenate([hx, hp, hm], axis=0)
            sT_ref[...] = sT
            whT_ref[...] = lax.dot_general(
                wg_ref[...], sT, (((0,), (0,)), ((), ())),
                preferred_element_type=jnp.float32)

    sT, whT = pl.pallas_call(
        feat_body,
        grid=(gn, nck),
        in_specs=[
            pl.BlockSpec((ic, bn), lambda i, j: (j, i)),
            pl.BlockSpec((dp, bn), lambda i, j: (0, i)),
            pl.BlockSpec((ic, nfc, bn), lambda i, j: (j, 0, i)),
            pl.BlockSpec((nfc, bn), lambda i, j: (0, i)),
            pl.BlockSpec((dp, nfc, bn), lambda i, j: (0, 0, i)),
            pl.BlockSpec((nfc, bn), lambda i, j: (0, i)),
            pl.BlockSpec((nfc, bn), lambda i, j: (0, i)),
            pl.BlockSpec((f, f), lambda i, j: (0, 0)),
        ],
        out_specs=[
            pl.BlockSpec((f, bn), lambda i, j: (0, i)),
            pl.BlockSpec((f, bn), lambda i, j: (0, i)),
        ],
        out_shape=[
            jax.ShapeDtypeStruct((f, n), jnp.float32),
            jax.ShapeDtypeStruct((f, n), jnp.float32),
        ],
        scratch_shapes=[
            pltpu.VMEM((nfc, bn), jnp.float32),
            pltpu.VMEM((nfc, bn), jnp.float32),
        ],
    )(xinT, pT, WxT, bxT, WpT, bpT, bmT, Wg)

    def head_body(sT_ref, whT_full_ref, whT_blk_ref, idxT_ref, a1_ref,
                  a2_ref, wiht_ref, bih_ref, waT_ref, baT_ref, out_ref):
        i = pl.program_id(0)
        whT_full = whT_full_ref[...]
        f2c = lax.dot_general(whT_full, a2_ref[...],
                              (((0,), (0,)), ((), ())),
                              preferred_element_type=jnp.float32)
        f1r = lax.dot_general(a1_ref[...], whT_blk_ref[...],
                              (((0,), (0,)), ((), ())),
                              preferred_element_type=jnp.float32)
        e = f2c + f1r
        e = jnp.where(e > 0, e, 0.2 * e)
        jsub = lax.broadcasted_iota(jnp.int32, (n, bn), 0)
        adj = jsub == (i * bn + lax.broadcasted_iota(jnp.int32, (n, bn), 1))
        for k in range(deg):
            adj = adj | (jsub == idxT_ref[k:k + 1, :])
        e = jnp.where(adj, e, jnp.float32(-9e15))
        m = jnp.max(e, axis=0, keepdims=True)
        ex = jnp.exp(e - m)
        att = ex / jnp.sum(ex, axis=0, keepdims=True)
        gat = lax.dot_general(whT_full, att, (((1,), (0,)), ((), ())),
                              preferred_element_type=jnp.float32)
        gat = jnp.where(gat > 0, gat, jnp.exp(gat) - 1.0)
        s2 = sT_ref[...] + gat
        gates = lax.dot_general(wiht_ref[...], s2, (((0,), (0,)), ((), ())),
                                preferred_element_type=jnp.float32)
        gates = gates + bih_ref[...]
        i_g = jax.nn.sigmoid(gates[:nh])
        g_g = jnp.tanh(gates[2 * nh:3 * nh])
        o_g = jax.nn.sigmoid(gates[3 * nh:4 * nh])
        h = o_g * jnp.tanh(i_g * g_g)
        acc = baT_ref[...].astype(jnp.float32)
        for r in range(nh):
            acc = acc + waT_ref[r] * h[r:r + 1, :]
        mx = jnp.max(acc, axis=0, keepdims=True)
        exl = jnp.exp(acc - mx)
        out_ref[...] = exl / jnp.sum(exl, axis=0, keepdims=True)

    probsT = pl.pallas_call(
        head_body,
        grid=(gn,),
        in_specs=[
            pl.BlockSpec((f, bn), lambda i: (0, i)),
            pl.BlockSpec((f, n), lambda i: (0, 0)),
            pl.BlockSpec((f, bn), lambda i: (0, i)),
            pl.BlockSpec((deg, bn), lambda i: (0, i)),
            pl.BlockSpec((f, 1), lambda i: (0, 0)),
            pl.BlockSpec((f, 1), lambda i: (0, 0)),
            pl.BlockSpec((f, 4 * nh), lambda i: (0, 0)),
            pl.BlockSpec((4 * nh, 1), lambda i: (0, 0)),
            pl.BlockSpec((nh, na, bn), lambda i: (0, 0, i)),
            pl.BlockSpec((na, bn), lambda i: (0, i)),
        ],
        out_specs=pl.BlockSpec((na, bn), lambda i: (0, i)),
        out_shape=jax.ShapeDtypeStruct((na, n), jnp.float32),
    )(sT, whT, whT, idxT, a1, a2, w_iht, bihC, WaT, baT)

    return probsT.T


# speedup vs baseline: 4.5074x; 1.0007x over previous
import functools

import jax
import jax.numpy as jnp
from jax import lax
from jax.experimental import pallas as pl
from jax.experimental.pallas import tpu as pltpu
from jax.experimental.pallas import tpu_sc as plsc


def _gather_rows(table, idx):
    _, d = table.shape
    b = idx.shape[0]
    info = plsc.get_sparse_core_info()
    nw = info.num_cores * info.num_subcores
    b_per_w = b // nw
    mesh = plsc.VectorSubcoreMesh(core_axis_name="c", subcore_axis_name="s")

    @functools.partial(
        pl.kernel,
        mesh=mesh,
        out_type=jax.ShapeDtypeStruct((b, d), jnp.float32),
        scratch_types=[
            pltpu.VMEM((b_per_w,), jnp.int32),
            pltpu.VMEM((b_per_w, d), jnp.float32),
            pltpu.SemaphoreType.DMA,
        ],
    )
    def gather_k(table_hbm, idx_hbm, out_hbm, idx_v, rows_v, sem):
        wid = lax.axis_index("s") * info.num_cores + lax.axis_index("c")
        base = wid * b_per_w
        pltpu.sync_copy(idx_hbm.at[pl.ds(base, b_per_w)], idx_v)
        pltpu.async_copy(table_hbm.at[idx_v], rows_v, sem).wait()
        pltpu.sync_copy(rows_v, out_hbm.at[pl.ds(base, b_per_w)])

    return gather_k(table, idx)


def kernel(ob, done, fp, neigh_idx, Wx, bx, Wp, bp, Wm, bm, Wg, a1, a2,
           W_ih, W_hh, b_ih, b_hh, Wa, ba):
    n, do = ob.shape
    na = fp.shape[1]
    deg = neigh_idx.shape[1]
    nfc = Wx.shape[2]
    nh = W_hh.shape[1]
    f = 3 * nfc
    dx = do * (deg + 1)
    dp = na * deg
    bn = 1024
    ic = 64
    nck = dx // ic
    gn = n // bn

    flat_idx = neigh_idx.reshape(-1).astype(jnp.int32)

    pad1 = (-(do + na)) % 128
    table1 = jnp.concatenate(
        [ob, fp, jnp.zeros((n, pad1), jnp.float32)], axis=1)
    g1 = _gather_rows(table1, flat_idx)

    nxT = g1[:, :do].reshape(n, deg * do).T
    pT = g1[:, do:do + na].reshape(n, deg * na).T
    WxT = Wx.transpose(1, 2, 0)
    WpT = Wp.transpose(1, 2, 0)
    WaT = Wa.transpose(1, 2, 0)
    bxT, bpT, bmT, baT = bx.T, bp.T, bm.T, ba.T
    idxT = neigh_idx.T.astype(jnp.int32)
    w_iht = W_ih.T
    bihC = (b_ih + b_hh).reshape(-1, 1)

    bna = 256

    def feat_a_body(obT_ref, wxT_ref, hxa_ref):
        acc = jnp.zeros((nfc, bna), jnp.float32)
        for r in range(ic):
            acc = acc + wxT_ref[r] * obT_ref[r:r + 1, :]
        hxa_ref[...] = acc

    hxa = pl.pallas_call(
        feat_a_body,
        grid=(n // bna,),
        in_specs=[
            pl.BlockSpec((ic, bna), lambda i: (0, i)),
            pl.BlockSpec((ic, nfc, bna), lambda i: (0, 0, i)),
        ],
        out_specs=pl.BlockSpec((nfc, bna), lambda i: (0, i)),
        out_shape=jax.ShapeDtypeStruct((nfc, n), jnp.float32),
    )(ob.T, WxT)

    def feat_body(nxT_ref, pT_ref, hxa_ref, wxT_ref, bxT_ref, wpT_ref,
                  bpT_ref, bmT_ref, wg_ref, sT_ref, whT_ref, accx, accp):
        j = pl.program_id(1)

        @pl.when(j == 0)
        def _init():
            acc = jnp.zeros((nfc, bn), jnp.float32)
            for r in range(dp):
                acc = acc + wpT_ref[r] * pT_ref[r:r + 1, :]
            accp[...] = acc
            accx[...] = hxa_ref[...]

        acc = accx[...]
        for r in range(ic):
            acc = acc + wxT_ref[r] * nxT_ref[r:r + 1, :]
        accx[...] = acc

        @pl.when(j == nck - 2)
        def _finalize():
            hx = jnp.maximum(accx[...] + bxT_ref[...], 0.0)
            hp = jnp.maximum(accp[...] + bpT_ref[...], 0.0)
            hm = jnp.maximum(bmT_ref[...], 0.0)
            sT = jnp.concatenate([hx, hp, hm], axis=0)
            sT_ref[...] = sT
            whT_ref[...] = lax.dot_general(
                wg_ref[...], sT, (((0,), (0,)), ((), ())),
                preferred_element_type=jnp.float32)

    sT, whT = pl.pallas_call(
        feat_body,
        grid=(gn, nck - 1),
        in_specs=[
            pl.BlockSpec((ic, bn), lambda i, j: (j, i)),
            pl.BlockSpec((dp, bn), lambda i, j: (0, i)),
            pl.BlockSpec((nfc, bn), lambda i, j: (0, i)),
            pl.BlockSpec((ic, nfc, bn), lambda i, j: (j + 1, 0, i)),
            pl.BlockSpec((nfc, bn), lambda i, j: (0, i)),
            pl.BlockSpec((dp, nfc, bn), lambda i, j: (0, 0, i)),
            pl.BlockSpec((nfc, bn), lambda i, j: (0, i)),
            pl.BlockSpec((nfc, bn), lambda i, j: (0, i)),
            pl.BlockSpec((f, f), lambda i, j: (0, 0)),
        ],
        out_specs=[
            pl.BlockSpec((f, bn), lambda i, j: (0, i)),
            pl.BlockSpec((f, bn), lambda i, j: (0, i)),
        ],
        out_shape=[
            jax.ShapeDtypeStruct((f, n), jnp.float32),
            jax.ShapeDtypeStruct((f, n), jnp.float32),
        ],
        scratch_shapes=[
            pltpu.VMEM((nfc, bn), jnp.float32),
            pltpu.VMEM((nfc, bn), jnp.float32),
        ],
    )(nxT, pT, hxa, WxT, bxT, WpT, bpT, bmT, Wg)

    def head_body(sT_ref, whT_full_ref, whT_blk_ref, idxT_ref, a1_ref,
                  a2_ref, wiht_ref, bih_ref, waT_ref, baT_ref, out_ref):
        i = pl.program_id(0)
        whT_full = whT_full_ref[...]
        f2c = lax.dot_general(whT_full, a2_ref[...],
                              (((0,), (0,)), ((), ())),
                              preferred_element_type=jnp.float32)
        f1r = lax.dot_general(a1_ref[...], whT_blk_ref[...],
                              (((0,), (0,)), ((), ())),
                              preferred_element_type=jnp.float32)
        e = f2c + f1r
        e = jnp.where(e > 0, e, 0.2 * e)
        jsub = lax.broadcasted_iota(jnp.int32, (n, bn), 0)
        adj = jsub == (i * bn + lax.broadcasted_iota(jnp.int32, (n, bn), 1))
        for k in range(deg):
            adj = adj | (jsub == idxT_ref[k:k + 1, :])
        e = jnp.where(adj, e, jnp.float32(-9e15))
        m = jnp.max(e, axis=0, keepdims=True)
        ex = jnp.exp(e - m)
        att = ex / jnp.sum(ex, axis=0, keepdims=True)
        gat = lax.dot_general(whT_full, att, (((1,), (0,)), ((), ())),
                              preferred_element_type=jnp.float32)
        gat = jnp.where(gat > 0, gat, jnp.exp(gat) - 1.0)
        s2 = sT_ref[...] + gat
        gates = lax.dot_general(wiht_ref[...], s2, (((0,), (0,)), ((), ())),
                                preferred_element_type=jnp.float32)
        gates = gates + bih_ref[...]
        i_g = jax.nn.sigmoid(gates[:nh])
        g_g = jnp.tanh(gates[2 * nh:3 * nh])
        o_g = jax.nn.sigmoid(gates[3 * nh:4 * nh])
        h = o_g * jnp.tanh(i_g * g_g)
        acc = baT_ref[...].astype(jnp.float32)
        for r in range(nh):
            acc = acc + waT_ref[r] * h[r:r + 1, :]
        mx = jnp.max(acc, axis=0, keepdims=True)
        exl = jnp.exp(acc - mx)
        out_ref[...] = exl / jnp.sum(exl, axis=0, keepdims=True)

    probsT = pl.pallas_call(
        head_body,
        grid=(gn,),
        in_specs=[
            pl.BlockSpec((f, bn), lambda i: (0, i)),
            pl.BlockSpec((f, n), lambda i: (0, 0)),
            pl.BlockSpec((f, bn), lambda i: (0, i)),
            pl.BlockSpec((deg, bn), lambda i: (0, i)),
            pl.BlockSpec((f, 1), lambda i: (0, 0)),
            pl.BlockSpec((f, 1), lambda i: (0, 0)),
            pl.BlockSpec((f, 4 * nh), lambda i: (0, 0)),
            pl.BlockSpec((4 * nh, 1), lambda i: (0, 0)),
            pl.BlockSpec((nh, na, bn), lambda i: (0, 0, i)),
            pl.BlockSpec((na, bn), lambda i: (0, i)),
        ],
        out_specs=pl.BlockSpec((na, bn), lambda i: (0, i)),
        out_shape=jax.ShapeDtypeStruct((na, n), jnp.float32),
    )(sT, whT, whT, idxT, a1, a2, w_iht, bihC, WaT, baT)

    return probsT.T


# speedup vs baseline: 4.7023x; 1.0432x over previous
import functools

import jax
import jax.numpy as jnp
from jax import lax
from jax.experimental import pallas as pl
from jax.experimental.pallas import tpu as pltpu
from jax.experimental.pallas import tpu_sc as plsc


def _gather_rows(table, idx):
    _, d = table.shape
    b = idx.shape[0]
    info = plsc.get_sparse_core_info()
    nw = info.num_cores * info.num_subcores
    b_per_w = b // nw
    mesh = plsc.VectorSubcoreMesh(core_axis_name="c", subcore_axis_name="s")

    @functools.partial(
        pl.kernel,
        mesh=mesh,
        out_type=jax.ShapeDtypeStruct((b, d), jnp.float32),
        scratch_types=[
            pltpu.VMEM((b_per_w,), jnp.int32),
            pltpu.VMEM((b_per_w, d), jnp.float32),
            pltpu.SemaphoreType.DMA,
        ],
    )
    def gather_k(table_hbm, idx_hbm, out_hbm, idx_v, rows_v, sem):
        wid = lax.axis_index("s") * info.num_cores + lax.axis_index("c")
        base = wid * b_per_w
        pltpu.sync_copy(idx_hbm.at[pl.ds(base, b_per_w)], idx_v)
        pltpu.async_copy(table_hbm.at[idx_v], rows_v, sem).wait()
        pltpu.sync_copy(rows_v, out_hbm.at[pl.ds(base, b_per_w)])

    return gather_k(table, idx)


def kernel(ob, done, fp, neigh_idx, Wx, bx, Wp, bp, Wm, bm, Wg, a1, a2,
           W_ih, W_hh, b_ih, b_hh, Wa, ba):
    n, do = ob.shape
    na = fp.shape[1]
    deg = neigh_idx.shape[1]
    nfc = Wx.shape[2]
    nh = W_hh.shape[1]
    f = 3 * nfc
    dx = do * (deg + 1)
    dp = na * deg
    bn = 1024
    ic = 64
    nck = dx // ic
    gn = n // bn

    idxT = neigh_idx.T.astype(jnp.int32)
    flat_idx = idxT.reshape(-1)

    pad1 = (-(do + na)) % 128
    table1 = jnp.concatenate(
        [ob, fp, jnp.zeros((n, pad1), jnp.float32)], axis=1)
    g1 = _gather_rows(table1, flat_idx)
    gT = g1.T.reshape(do + pad1 + na, deg, 1, n)

    WxT = Wx.transpose(1, 2, 0)
    WpT = Wp.transpose(1, 2, 0)
    WaT = Wa.transpose(1, 2, 0)
    bxT, bpT, bmT, baT = bx.T, bp.T, bm.T, ba.T
    w_iht = W_ih.T
    bihC = (b_ih + b_hh).reshape(-1, 1)

    bna = 512

    def feat_a_body(obT_ref, wxT_ref, hxa_ref):
        acc = jnp.zeros((nfc, bna), jnp.float32)
        for r in range(ic):
            acc = acc + wxT_ref[r] * obT_ref[r:r + 1, :]
        hxa_ref[...] = acc

    hxa = pl.pallas_call(
        feat_a_body,
        grid=(n // bna,),
        in_specs=[
            pl.BlockSpec((ic, bna), lambda i: (0, i)),
            pl.BlockSpec((ic, nfc, bna), lambda i: (0, 0, i)),
        ],
        out_specs=pl.BlockSpec((nfc, bna), lambda i: (0, i)),
        out_shape=jax.ShapeDtypeStruct((nfc, n), jnp.float32),
    )(ob.T, WxT)

    def feat_body(gx_ref, gp_ref, hxa_ref, wxT_ref, bxT_ref, wpT_ref,
                  bpT_ref, bmT_ref, wg_ref, sT_ref, whT_ref, accx, accp):
        j = pl.program_id(1)

        @pl.when(j == 0)
        def _init():
            accp[...] = jnp.zeros((nfc, bn), jnp.float32)
            accx[...] = hxa_ref[...]

        acc = accx[...]
        for r in range(ic):
            acc = acc + wxT_ref[r] * gx_ref[r:r + 1, 0, 0, :]
        accx[...] = acc
        accq = accp[...]
        for r in range(na):
            accq = accq + wpT_ref[r] * gp_ref[r:r + 1, 0, 0, :]
        accp[...] = accq

        @pl.when(j == deg - 1)
        def _finalize():
            hx = jnp.maximum(accx[...] + bxT_ref[...], 0.0)
            hp = jnp.maximum(accp[...] + bpT_ref[...], 0.0)
            hm = jnp.maximum(bmT_ref[...], 0.0)
            sT = jnp.concatenate([hx, hp, hm], axis=0)
            sT_ref[...] = sT
            whT_ref[...] = lax.dot_general(
                wg_ref[...], sT, (((0,), (0,)), ((), ())),
                preferred_element_type=jnp.float32)

    sT, whT = pl.pallas_call(
        feat_body,
        grid=(gn, deg),
        in_specs=[
            pl.BlockSpec((do, 1, 1, bn), lambda i, j: (0, j, 0, i)),
            pl.BlockSpec((na, 1, 1, bn), lambda i, j: (do // na, j, 0, i)),
            pl.BlockSpec((nfc, bn), lambda i, j: (0, i)),
            pl.BlockSpec((ic, nfc, bn), lambda i, j: (j + 1, 0, i)),
            pl.BlockSpec((nfc, bn), lambda i, j: (0, i)),
            pl.BlockSpec((na, nfc, bn), lambda i, j: (j, 0, i)),
            pl.BlockSpec((nfc, bn), lambda i, j: (0, i)),
            pl.BlockSpec((nfc, bn), lambda i, j: (0, i)),
            pl.BlockSpec((f, f), lambda i, j: (0, 0)),
        ],
        out_specs=[
            pl.BlockSpec((f, bn), lambda i, j: (0, i)),
            pl.BlockSpec((f, bn), lambda i, j: (0, i)),
        ],
        out_shape=[
            jax.ShapeDtypeStruct((f, n), jnp.float32),
            jax.ShapeDtypeStruct((f, n), jnp.float32),
        ],
        scratch_shapes=[
            pltpu.VMEM((nfc, bn), jnp.float32),
            pltpu.VMEM((nfc, bn), jnp.float32),
        ],
    )(gT, gT, hxa, WxT, bxT, WpT, bpT, bmT, Wg)

    def head_body(sT_ref, whT_full_ref, whT_blk_ref, idxT_ref, a1_ref,
                  a2_ref, wiht_ref, bih_ref, waT_ref, baT_ref, out_ref):
        i = pl.program_id(0)
        whT_full = whT_full_ref[...]
        f2c = lax.dot_general(whT_full, a2_ref[...],
                              (((0,), (0,)), ((), ())),
                              preferred_element_type=jnp.float32)
        f1r = lax.dot_general(a1_ref[...], whT_blk_ref[...],
                              (((0,), (0,)), ((), ())),
                              preferred_element_type=jnp.float32)
        e = f2c + f1r
        e = jnp.where(e > 0, e, 0.2 * e)
        jsub = lax.broadcasted_iota(jnp.int32, (n, bn), 0)
        adj = jsub == (i * bn + lax.broadcasted_iota(jnp.int32, (n, bn), 1))
        for k in range(deg):
            adj = adj | (jsub == idxT_ref[k:k + 1, :])
        e = jnp.where(adj, e, jnp.float32(-9e15))
        m = jnp.max(e, axis=0, keepdims=True)
        ex = jnp.exp(e - m)
        att = ex / jnp.sum(ex, axis=0, keepdims=True)
        gat = lax.dot_general(whT_full, att, (((1,), (0,)), ((), ())),
                              preferred_element_type=jnp.float32)
        gat = jnp.where(gat > 0, gat, jnp.exp(gat) - 1.0)
        s2 = sT_ref[...] + gat
        gates = lax.dot_general(wiht_ref[...], s2, (((0,), (0,)), ((), ())),
                                preferred_element_type=jnp.float32)
        gates = gates + bih_ref[...]
        i_g = jax.nn.sigmoid(gates[:nh])
        g_g = jnp.tanh(gates[2 * nh:3 * nh])
        o_g = jax.nn.sigmoid(gates[3 * nh:4 * nh])
        h = o_g * jnp.tanh(i_g * g_g)
        acc = baT_ref[...].astype(jnp.float32)
        for r in range(nh):
            acc = acc + waT_ref[r] * h[r:r + 1, :]
        mx = jnp.max(acc, axis=0, keepdims=True)
        exl = jnp.exp(acc - mx)
        out_ref[...] = exl / jnp.sum(exl, axis=0, keepdims=True)

    probsT = pl.pallas_call(
        head_body,
        grid=(gn,),
        in_specs=[
            pl.BlockSpec((f, bn), lambda i: (0, i)),
            pl.BlockSpec((f, n), lambda i: (0, 0)),
            pl.BlockSpec((f, bn), lambda i: (0, i)),
            pl.BlockSpec((deg, bn), lambda i: (0, i)),
            pl.BlockSpec((f, 1), lambda i: (0, 0)),
            pl.BlockSpec((f, 1), lambda i: (0, 0)),
            pl.BlockSpec((f, 4 * nh), lambda i: (0, 0)),
            pl.BlockSpec((4 * nh, 1), lambda i: (0, 0)),
            pl.BlockSpec((nh, na, bn), lambda i: (0, 0, i)),
            pl.BlockSpec((na, bn), lambda i: (0, i)),
        ],
        out_specs=pl.BlockSpec((na, bn), lambda i: (0, i)),
        out_shape=jax.ShapeDtypeStruct((na, n), jnp.float32),
    )(sT, whT, whT, idxT, a1, a2, w_iht, bihC, WaT, baT)

    return probsT.T


# speedup vs baseline: 5.1596x; 1.0973x over previous
import functools

import jax
import jax.numpy as jnp
from jax import lax
from jax.experimental import pallas as pl
from jax.experimental.pallas import tpu as pltpu
from jax.experimental.pallas import tpu_sc as plsc


def _gather_rows(table, idx):
    _, d = table.shape
    b = idx.shape[0]
    info = plsc.get_sparse_core_info()
    nw = info.num_cores * info.num_subcores
    b_per_w = b // nw
    mesh = plsc.VectorSubcoreMesh(core_axis_name="c", subcore_axis_name="s")

    @functools.partial(
        pl.kernel,
        mesh=mesh,
        out_type=jax.ShapeDtypeStruct((b, d), jnp.float32),
        scratch_types=[
            pltpu.VMEM((b_per_w,), jnp.int32),
            pltpu.VMEM((b_per_w, d), jnp.float32),
            pltpu.SemaphoreType.DMA,
        ],
    )
    def gather_k(table_hbm, idx_hbm, out_hbm, idx_v, rows_v, sem):
        wid = lax.axis_index("s") * info.num_cores + lax.axis_index("c")
        base = wid * b_per_w
        pltpu.sync_copy(idx_hbm.at[pl.ds(base, b_per_w)], idx_v)
        pltpu.async_copy(table_hbm.at[idx_v], rows_v, sem).wait()
        pltpu.sync_copy(rows_v, out_hbm.at[pl.ds(base, b_per_w)])

    return gather_k(table, idx)


def kernel(ob, done, fp, neigh_idx, Wx, bx, Wp, bp, Wm, bm, Wg, a1, a2,
           W_ih, W_hh, b_ih, b_hh, Wa, ba):
    n, do = ob.shape
    na = fp.shape[1]
    deg = neigh_idx.shape[1]
    nfc = Wx.shape[2]
    nh = W_hh.shape[1]
    f = 3 * nfc
    dx = do * (deg + 1)
    dp = na * deg
    bn = 1024
    ic = 64
    nck = dx // ic
    gn = n // bn

    idxT = neigh_idx.T.astype(jnp.int32)
    flat_idx = idxT.reshape(-1)

    pad1 = (-(do + na)) % 128
    table1 = jnp.concatenate(
        [ob, fp, jnp.zeros((n, pad1), jnp.float32)], axis=1)
    g1 = _gather_rows(table1, flat_idx)

    WxT = Wx.transpose(1, 2, 0)
    WpT = Wp.transpose(1, 2, 0)
    WaT = Wa.transpose(1, 2, 0)
    bxT, bpT, bmT, baT = bx.T, bp.T, bm.T, ba.T
    w_iht = W_ih.T
    bihC = (b_ih + b_hh).reshape(-1, 1)

    bna = 512

    def feat_a_body(obT_ref, wxT_ref, hxa_ref):
        acc = jnp.zeros((nfc, bna), jnp.float32)
        for r in range(ic):
            acc = acc + wxT_ref[r] * obT_ref[r:r + 1, :]
        hxa_ref[...] = acc

    hxa = pl.pallas_call(
        feat_a_body,
        grid=(n // bna,),
        in_specs=[
            pl.BlockSpec((ic, bna), lambda i: (0, i)),
            pl.BlockSpec((ic, nfc, bna), lambda i: (0, 0, i)),
        ],
        out_specs=pl.BlockSpec((nfc, bna), lambda i: (0, i)),
        out_shape=jax.ShapeDtypeStruct((nfc, n), jnp.float32),
    )(ob.T, WxT)

    def feat_body(g_ref, hxa_ref, wxT_ref, bxT_ref, wpT_ref,
                  bpT_ref, bmT_ref, wg_ref, sT_ref, whT_ref, accx, accp):
        j = pl.program_id(1)

        @pl.when(j == 0)
        def _init():
            accp[...] = jnp.zeros((nfc, bn), jnp.float32)
            accx[...] = hxa_ref[...]

        gt = g_ref[...].T
        acc = accx[...]
        for r in range(ic):
            acc = acc + wxT_ref[r] * gt[r:r + 1, :]
        accx[...] = acc
        accq = accp[...]
        for r in range(na):
            accq = accq + wpT_ref[r] * gt[do + r:do + r + 1, :]
        accp[...] = accq

        @pl.when(j == deg - 1)
        def _finalize():
            hx = jnp.maximum(accx[...] + bxT_ref[...], 0.0)
            hp = jnp.maximum(accp[...] + bpT_ref[...], 0.0)
            hm = jnp.maximum(bmT_ref[...], 0.0)
            sT = jnp.concatenate([hx, hp, hm], axis=0)
            sT_ref[...] = sT
            whT_ref[...] = lax.dot_general(
                wg_ref[...], sT, (((0,), (0,)), ((), ())),
                preferred_element_type=jnp.float32)

    sT, whT = pl.pallas_call(
        feat_body,
        grid=(gn, deg),
        in_specs=[
            pl.BlockSpec((bn, do + pad1 + na),
                         lambda i, j: (j * gn + i, 0)),
            pl.BlockSpec((nfc, bn), lambda i, j: (0, i)),
            pl.BlockSpec((ic, nfc, bn), lambda i, j: (j + 1, 0, i)),
            pl.BlockSpec((nfc, bn), lambda i, j: (0, i)),
            pl.BlockSpec((na, nfc, bn), lambda i, j: (j, 0, i)),
            pl.BlockSpec((nfc, bn), lambda i, j: (0, i)),
            pl.BlockSpec((nfc, bn), lambda i, j: (0, i)),
            pl.BlockSpec((f, f), lambda i, j: (0, 0)),
        ],
        out_specs=[
            pl.BlockSpec((f, bn), lambda i, j: (0, i)),
            pl.BlockSpec((f, bn), lambda i, j: (0, i)),
        ],
        out_shape=[
            jax.ShapeDtypeStruct((f, n), jnp.float32),
            jax.ShapeDtypeStruct((f, n), jnp.float32),
        ],
        scratch_shapes=[
            pltpu.VMEM((nfc, bn), jnp.float32),
            pltpu.VMEM((nfc, bn), jnp.float32),
        ],
    )(g1, hxa, WxT, bxT, WpT, bpT, bmT, Wg)

    def head_body(sT_ref, whT_full_ref, whT_blk_ref, idxT_ref, a1_ref,
                  a2_ref, wiht_ref, bih_ref, waT_ref, baT_ref, out_ref):
        i = pl.program_id(0)
        whT_full = whT_full_ref[...]
        f2c = lax.dot_general(whT_full, a2_ref[...],
                              (((0,), (0,)), ((), ())),
                              preferred_element_type=jnp.float32)
        f1r = lax.dot_general(a1_ref[...], whT_blk_ref[...],
                              (((0,), (0,)), ((), ())),
                              preferred_element_type=jnp.float32)
        e = f2c + f1r
        e = jnp.where(e > 0, e, 0.2 * e)
        jsub = lax.broadcasted_iota(jnp.int32, (n, bn), 0)
        adj = jsub == (i * bn + lax.broadcasted_iota(jnp.int32, (n, bn), 1))
        for k in range(deg):
            adj = adj | (jsub == idxT_ref[k:k + 1, :])
        e = jnp.where(adj, e, jnp.float32(-9e15))
        m = jnp.max(e, axis=0, keepdims=True)
        ex = jnp.exp(e - m)
        att = ex / jnp.sum(ex, axis=0, keepdims=True)
        gat = lax.dot_general(whT_full, att, (((1,), (0,)), ((), ())),
                              preferred_element_type=jnp.float32)
        gat = jnp.where(gat > 0, gat, jnp.exp(gat) - 1.0)
        s2 = sT_ref[...] + gat
        gates = lax.dot_general(wiht_ref[...], s2, (((0,), (0,)), ((), ())),
                                preferred_element_type=jnp.float32)
        gates = gates + bih_ref[...]
        i_g = jax.nn.sigmoid(gates[:nh])
        g_g = jnp.tanh(gates[2 * nh:3 * nh])
        o_g = jax.nn.sigmoid(gates[3 * nh:4 * nh])
        h = o_g * jnp.tanh(i_g * g_g)
        acc = baT_ref[...].astype(jnp.float32)
        for r in range(nh):
            acc = acc + waT_ref[r] * h[r:r + 1, :]
        mx = jnp.max(acc, axis=0, keepdims=True)
        exl = jnp.exp(acc - mx)
        out_ref[...] = exl / jnp.sum(exl, axis=0, keepdims=True)

    probsT = pl.pallas_call(
        head_body,
        grid=(gn,),
        in_specs=[
            pl.BlockSpec((f, bn), lambda i: (0, i)),
            pl.BlockSpec((f, n), lambda i: (0, 0)),
            pl.BlockSpec((f, bn), lambda i: (0, i)),
            pl.BlockSpec((deg, bn), lambda i: (0, i)),
            pl.BlockSpec((f, 1), lambda i: (0, 0)),
            pl.BlockSpec((f, 1), lambda i: (0, 0)),
            pl.BlockSpec((f, 4 * nh), lambda i: (0, 0)),
            pl.BlockSpec((4 * nh, 1), lambda i: (0, 0)),
            pl.BlockSpec((nh, na, bn), lambda i: (0, 0, i)),
            pl.BlockSpec((na, bn), lambda i: (0, i)),
        ],
        out_specs=pl.BlockSpec((na, bn), lambda i: (0, i)),
        out_shape=jax.ShapeDtypeStruct((na, n), jnp.float32),
    )(sT, whT, whT, idxT, a1, a2, w_iht, bihC, WaT, baT)

    return probsT.T
